# baseline (speedup 1.0000x reference)
import functools

import jax
import jax.numpy as jnp
from jax.experimental import pallas as pl


def _conv_kern(x_ref, w_ref, b_ref, o_ref, *, H, W, taps, mode):
    x = x_ref[0]
    cin = x.shape[-1]
    acc = None
    for t, (dh, dw) in enumerate(taps):
        sl = jax.lax.slice(x, (dh, dw, 0), (dh + H, dw + W, cin))
        sl2 = sl.reshape(H * W, cin)
        p = jax.lax.dot_general(sl2, w_ref[t], (((1,), (0,)), ((), ())),
                                preferred_element_type=jnp.float32)
        acc = p if acc is None else acc + p
    acc = jnp.maximum(acc + b_ref[:], 0.0)
    o_ref[0] = jnp.sum(acc, axis=0, keepdims=True) * (1.0 / (H * W))


def _conv_pool_kern(x_ref, w_ref, b_ref, o_ref, *, Ho, Wo, taps):
    cin = x_ref.shape[-1]
    m = None
    for rh in (0, 1):
        for rw in (0, 1):
            acc = None
            for t, (dh, dw) in enumerate(taps):
                s, oh = (rh + dh) % 2, (rh + dh) // 2
                u, ow = (rw + dw) % 2, (rw + dw) // 2
                sl = jax.lax.slice(x_ref[0, s, u], (oh, ow, 0),
                                   (oh + Ho, ow + Wo, cin))
                p = jax.lax.dot_general(sl.reshape(Ho * Wo, cin), w_ref[t],
                                        (((1,), (0,)), ((), ())),
                                        preferred_element_type=jnp.float32)
                acc = p if acc is None else acc + p
            m = acc if m is None else jnp.maximum(m, acc)
    m = jnp.maximum(m + b_ref[:], 0.0)
    o_ref[0] = m.reshape(Ho, Wo, -1)


def _conv_pool_layer(xq, wcol, brow, *, Ho, Wo, taps, ht=None):
    B = xq.shape[0]
    cout = wcol.shape[-1]
    if ht is None:
        ht = Ho
    grid = (B, Ho // ht)
    hq_blk = ht + (1 if len(taps) > 1 else 0)
    if hq_blk > xq.shape[3]:
        hq_blk = xq.shape[3]
    return pl.pallas_call(
        functools.partial(_conv_pool_kern, Ho=ht, Wo=Wo, taps=taps),
        grid=grid,
        in_specs=[
            pl.BlockSpec((1, 2, 2, hq_blk) + xq.shape[4:],
                         lambda b, i: (b, 0, 0, i, 0, 0)),
            pl.BlockSpec(wcol.shape, lambda b, i: (0, 0, 0)),
            pl.BlockSpec(brow.shape, lambda b, i: (0, 0)),
        ],
        out_specs=pl.BlockSpec((1, ht, Wo, cout), lambda b, i: (b, i, 0, 0)),
        out_shape=jax.ShapeDtypeStruct((B, Ho, Wo, cout), jnp.float32),
    )(xq, wcol, brow)


def _quadrants(xp):
    rows = [[xp[:, s::2, t::2, :] for t in (0, 1)] for s in (0, 1)]
    return jnp.stack([jnp.stack(r, axis=1) for r in rows], axis=1)


def _conv_layer(x, wcol, brow, *, H, W, taps, mode, ht=None):
    B = x.shape[0]
    cout = wcol.shape[-1]
    if ht is None:
        ht = H
    grid = (B, H // ht)
    hpad = x.shape[1] - H
    if mode == 'mean':
        out_shape = jax.ShapeDtypeStruct((B, 1, cout), jnp.float32)
        out_spec = pl.BlockSpec((1, 1, cout), lambda b, i: (b, 0, 0))
    else:
        out_shape = jax.ShapeDtypeStruct((B, H // 2, W // 2, cout), jnp.float32)
        out_spec = pl.BlockSpec((1, ht // 2, W // 2, cout),
                                lambda b, i: (b, i, 0, 0))
    return pl.pallas_call(
        functools.partial(_conv_kern, H=ht, W=W, taps=taps, mode=mode),
        grid=grid,
        in_specs=[
            pl.BlockSpec((1, ht + hpad) + x.shape[2:],
                         lambda b, i: (b, i, 0, 0)),
            pl.BlockSpec(wcol.shape, lambda b, i: (0, 0, 0)),
            pl.BlockSpec(brow.shape, lambda b, i: (0, 0)),
        ],
        out_specs=out_spec,
        out_shape=out_shape,
    )(x, wcol, brow)


def _heads_kern(gf_ref, n1w_ref, n1b_ref, n2w_ref, n2b_ref,
                c1w_ref, c1b_ref, c2w_ref, c2b_ref, cf_ref, cnt_ref):
    gf = gf_ref[:]
    dn = (((1,), (1,)), ((), ()))
    nh = jnp.maximum(
        jax.lax.dot_general(gf, n1w_ref[:], dn,
                            preferred_element_type=jnp.float32) + n1b_ref[:], 0.0)
    cf_ref[:] = jax.lax.dot_general(nh, n2w_ref[:], dn,
                                    preferred_element_type=jnp.float32) + n2b_ref[:]
    ch = jnp.maximum(
        jax.lax.dot_general(gf, c1w_ref[:], dn,
                            preferred_element_type=jnp.float32) + c1b_ref[:], 0.0)
    cnt_ref[:] = jax.lax.dot_general(ch, c2w_ref[:], dn,
                                     preferred_element_type=jnp.float32) + c2b_ref[:]


def _edge_kern(c_ref, w1_ref, b1_ref, w2_ref, b2_ref, o_ref, *, N):
    c = c_ref[0]
    w1 = w1_ref[:]
    wa = jax.lax.slice(w1, (0, 0), (64, 2))
    wb = jax.lax.slice(w1, (0, 2), (64, 4))
    u = jax.lax.dot_general(c, wa, (((1,), (1,)), ((), ())),
                            preferred_element_type=jnp.float32) + b1_ref[:]
    vT = jax.lax.dot_general(wb, c, (((1,), (1,)), ((), ())),
                             preferred_element_type=jnp.float32)
    E = jnp.zeros((N, N), jnp.float32)
    for k in range(64):
        t = jnp.maximum(u[:, k:k + 1] + vT[k:k + 1, :], 0.0)
        E = E + t * w2_ref[0:1, k:k + 1]
    e = jax.nn.sigmoid(E + b2_ref[0:1, 0:1])
    row = jax.lax.broadcasted_iota(jnp.int32, (N, N), 0)
    col = jax.lax.broadcasted_iota(jnp.int32, (N, N), 1)
    eu = jnp.where(row < col, e, 0.0)
    o_ref[0] = eu + eu.T


_TAPS9 = tuple((dh, dw) for dh in range(3) for dw in range(3))


def kernel(images, node_masks, c1_w, c1_b, c2_w, c2_b, c3_w, c3_b, c4_w, c4_b,
           np1_w, np1_b, np2_w, np2_b, cp1_w, cp1_b, cp2_w, cp2_b,
           ep1_w, ep1_b, ep2_w, ep2_b):
    B = images.shape[0]
    N = 350

    def col9(w):
        return jnp.transpose(w, (2, 3, 1, 0)).reshape(9, w.shape[1], w.shape[0])

    w1col = jnp.transpose(c1_w[:, 0], (1, 2, 0)).reshape(1, 9, 32)
    w2col, w3col, w4col = col9(c2_w), col9(c3_w), col9(c4_w)
    b1r, b2r, b3r, b4r = (b.reshape(1, -1) for b in (c1_b, c2_b, c3_b, c4_b))

    x = jnp.pad(images[:, 0], ((0, 0), (1, 1), (1, 1)))
    patches = jnp.stack(
        [x[:, dh:dh + 224, dw:dw + 224] for dh, dw in _TAPS9], axis=-1)
    pq = _quadrants(patches)
    h = _conv_pool_layer(pq, w1col, b1r, Ho=112, Wo=112, taps=((0, 0),), ht=28)
    hq = _quadrants(jnp.pad(h, ((0, 0), (1, 1), (1, 1), (0, 0))))
    h = _conv_pool_layer(hq, w2col, b2r, Ho=56, Wo=56, taps=_TAPS9)
    hq = _quadrants(jnp.pad(h, ((0, 0), (1, 1), (1, 1), (0, 0))))
    h = _conv_pool_layer(hq, w3col, b3r, Ho=28, Wo=28, taps=_TAPS9)
    h = jnp.pad(h, ((0, 0), (1, 1), (1, 1), (0, 0)))
    gf = _conv_layer(h, w4col, b4r, H=28, W=28, taps=_TAPS9, mode='mean')
    gf = gf.reshape(B, 256)

    coords_flat, cnt = pl.pallas_call(
        _heads_kern,
        grid=(1,),
        in_specs=[pl.BlockSpec(s, lambda g: (0, 0)) for s in
                  ((B, 256), (512, 256), (1, 512), (700, 512), (1, 700),
                   (256, 256), (1, 256), (128, 256), (1, 128))],
        out_specs=[pl.BlockSpec((B, 700), lambda g: (0, 0)),
                   pl.BlockSpec((B, 128), lambda g: (0, 0))],
        out_shape=[jax.ShapeDtypeStruct((B, 700), jnp.float32),
                   jax.ShapeDtypeStruct((B, 128), jnp.float32)],
    )(gf, np1_w, np1_b.reshape(1, 512), np2_w, np2_b.reshape(1, 700),
      cp1_w, cp1_b.reshape(1, 256), jnp.pad(cp2_w, ((0, 127), (0, 0))),
      jnp.pad(cp2_b.reshape(1, 1), ((0, 0), (0, 127))))

    coords = coords_flat.reshape(B, N, 2)

    adj = pl.pallas_call(
        functools.partial(_edge_kern, N=N),
        grid=(B,),
        in_specs=[
            pl.BlockSpec((1, N, 2), lambda b: (b, 0, 0)),
            pl.BlockSpec((64, 4), lambda b: (0, 0)),
            pl.BlockSpec((1, 64), lambda b: (0, 0)),
            pl.BlockSpec((1, 64), lambda b: (0, 0)),
            pl.BlockSpec((1, 1), lambda b: (0, 0)),
        ],
        out_specs=pl.BlockSpec((1, N, N), lambda b: (b, 0, 0)),
        out_shape=jax.ShapeDtypeStruct((B, N, N), jnp.float32),
    )(coords, ep1_w, ep1_b.reshape(1, 64), ep2_w, ep2_b.reshape(1, 1))

    return coords, adj, cnt[:, 0]


# speedup vs baseline: 12.6812x; 12.6812x over previous
import functools

import jax
import jax.numpy as jnp
from jax.experimental import pallas as pl

_TAPS9 = tuple((dh, dw) for dh in range(3) for dw in range(3))


def _conv1_kern(x_ref, w_ref, b_ref, o_ref, *, ht, W):
    i = pl.program_id(1)
    xt = x_ref[0, pl.ds(i * ht, ht + 2), :]
    sls = [jax.lax.slice(xt, (dh, dw), (dh + ht, dw + W)) for dh, dw in _TAPS9]
    P = jnp.stack(sls, axis=0)
    acc = jax.lax.dot_general(P, w_ref[0], (((0,), (0,)), ((), ())),
                              preferred_element_type=jnp.float32)
    acc = jnp.maximum(acc + b_ref[:], 0.0)
    a = jnp.max(acc.reshape(ht // 2, 2, W, 32), axis=1)
    a = jnp.max(a.reshape(ht // 2, W // 2, 2, 32), axis=2)
    o_ref[0] = a


def _conv_kern(x_ref, w_ref, b_ref, o_ref, *, H, W, mode):
    x = x_ref[0]
    cin = x.shape[-1]
    acc = None
    for t, (dh, dw) in enumerate(_TAPS9):
        sl = jax.lax.slice(x, (dh, dw, 0), (dh + H, dw + W, cin))
        p = jax.lax.dot_general(sl.reshape(H * W, cin), w_ref[t],
                                (((1,), (0,)), ((), ())),
                                preferred_element_type=jnp.float32)
        acc = p if acc is None else acc + p
    acc = jnp.maximum(acc + b_ref[:], 0.0)
    if mode == 'mean':
        o_ref[0] = jnp.sum(acc, axis=0, keepdims=True) * (1.0 / (H * W))
    else:
        cout = acc.shape[-1]
        a = jnp.max(acc.reshape(H // 2, 2, W, cout), axis=1)
        a = jnp.max(a.reshape(H // 2, W // 2, 2, cout), axis=2)
        o_ref[0] = a


def _conv_layer(x, wcol, brow, *, H, W, mode):
    B = x.shape[0]
    cout = wcol.shape[-1]
    if mode == 'mean':
        out_shape = jax.ShapeDtypeStruct((B, 1, cout), jnp.float32)
        out_spec = pl.BlockSpec((1, 1, cout), lambda b: (b, 0, 0))
    else:
        out_shape = jax.ShapeDtypeStruct((B, H // 2, W // 2, cout), jnp.float32)
        out_spec = pl.BlockSpec((1, H // 2, W // 2, cout), lambda b: (b, 0, 0, 0))
    return pl.pallas_call(
        functools.partial(_conv_kern, H=H, W=W, mode=mode),
        grid=(B,),
        in_specs=[
            pl.BlockSpec((1,) + x.shape[1:], lambda b: (b, 0, 0, 0)),
            pl.BlockSpec(wcol.shape, lambda b: (0, 0, 0)),
            pl.BlockSpec(brow.shape, lambda b: (0, 0)),
        ],
        out_specs=out_spec,
        out_shape=out_shape,
    )(x, wcol, brow)


def _heads_kern(gf_ref, n1w_ref, n1b_ref, n2w_ref, n2b_ref,
                c1w_ref, c1b_ref, c2w_ref, c2b_ref, cf_ref, cnt_ref):
    gf = gf_ref[:]
    dn = (((1,), (1,)), ((), ()))
    nh = jnp.maximum(
        jax.lax.dot_general(gf, n1w_ref[:], dn,
                            preferred_element_type=jnp.float32) + n1b_ref[:], 0.0)
    cf_ref[:] = jax.lax.dot_general(nh, n2w_ref[:], dn,
                                    preferred_element_type=jnp.float32) + n2b_ref[:]
    ch = jnp.maximum(
        jax.lax.dot_general(gf, c1w_ref[:], dn,
                            preferred_element_type=jnp.float32) + c1b_ref[:], 0.0)
    cnt_ref[:] = jax.lax.dot_general(ch, c2w_ref[:], dn,
                                     preferred_element_type=jnp.float32) + c2b_ref[:]


def _edge_kern(c_ref, w1_ref, b1_ref, w2_ref, b2_ref, o_ref, *, N):
    c = c_ref[0]
    w1 = w1_ref[:]
    wa = jax.lax.slice(w1, (0, 0), (64, 2))
    wb = jax.lax.slice(w1, (0, 2), (64, 4))
    u = jax.lax.dot_general(c, wa, (((1,), (1,)), ((), ())),
                            preferred_element_type=jnp.float32) + b1_ref[:]
    vT = jax.lax.dot_general(wb, c, (((1,), (1,)), ((), ())),
                             preferred_element_type=jnp.float32)
    E = jnp.zeros((N, N), jnp.float32)
    for k in range(64):
        t = jnp.maximum(u[:, k:k + 1] + vT[k:k + 1, :], 0.0)
        E = E + t * w2_ref[0:1, k:k + 1]
    e = jax.nn.sigmoid(E + b2_ref[0:1, 0:1])
    row = jax.lax.broadcasted_iota(jnp.int32, (N, N), 0)
    col = jax.lax.broadcasted_iota(jnp.int32, (N, N), 1)
    eu = jnp.where(row < col, e, 0.0)
    o_ref[0] = eu + eu.T


def kernel(images, node_masks, c1_w, c1_b, c2_w, c2_b, c3_w, c3_b, c4_w, c4_b,
           np1_w, np1_b, np2_w, np2_b, cp1_w, cp1_b, cp2_w, cp2_b,
           ep1_w, ep1_b, ep2_w, ep2_b):
    B = images.shape[0]
    N = 350

    def col9(w):
        return jnp.transpose(w, (2, 3, 1, 0)).reshape(9, w.shape[1], w.shape[0])

    w1col = jnp.transpose(c1_w[:, 0], (1, 2, 0)).reshape(1, 9, 32)
    w2col, w3col, w4col = col9(c2_w), col9(c3_w), col9(c4_w)
    b1r, b2r, b3r, b4r = (b.reshape(1, -1) for b in (c1_b, c2_b, c3_b, c4_b))

    xpad = jnp.pad(images[:, 0], ((0, 0), (1, 1), (1, 1)))
    ht = 32
    h = pl.pallas_call(
        functools.partial(_conv1_kern, ht=ht, W=224),
        grid=(B, 224 // ht),
        in_specs=[
            pl.BlockSpec((1, 226, 226), lambda b, i: (b, 0, 0)),
            pl.BlockSpec((1, 9, 32), lambda b, i: (0, 0, 0)),
            pl.BlockSpec((1, 32), lambda b, i: (0, 0)),
        ],
        out_specs=pl.BlockSpec((1, ht // 2, 112, 32), lambda b, i: (b, i, 0, 0)),
        out_shape=jax.ShapeDtypeStruct((B, 112, 112, 32), jnp.float32),
    )(xpad, w1col, b1r)

    h = jnp.pad(h, ((0, 0), (1, 1), (1, 1), (0, 0)))
    h = _conv_layer(h, w2col, b2r, H=112, W=112, mode='pool')
    h = jnp.pad(h, ((0, 0), (1, 1), (1, 1), (0, 0)))
    h = _conv_layer(h, w3col, b3r, H=56, W=56, mode='pool')
    h = jnp.pad(h, ((0, 0), (1, 1), (1, 1), (0, 0)))
    gf = _conv_layer(h, w4col, b4r, H=28, W=28, mode='mean')
    gf = gf.reshape(B, 256)

    coords_flat, cnt = pl.pallas_call(
        _heads_kern,
        grid=(1,),
        in_specs=[pl.BlockSpec(s, lambda g: (0, 0)) for s in
                  ((B, 256), (512, 256), (1, 512), (700, 512), (1, 700),
                   (256, 256), (1, 256), (128, 256), (1, 128))],
        out_specs=[pl.BlockSpec((B, 700), lambda g: (0, 0)),
                   pl.BlockSpec((B, 128), lambda g: (0, 0))],
        out_shape=[jax.ShapeDtypeStruct((B, 700), jnp.float32),
                   jax.ShapeDtypeStruct((B, 128), jnp.float32)],
    )(gf, np1_w, np1_b.reshape(1, 512), np2_w, np2_b.reshape(1, 700),
      cp1_w, cp1_b.reshape(1, 256), jnp.pad(cp2_w, ((0, 127), (0, 0))),
      jnp.pad(cp2_b.reshape(1, 1), ((0, 0), (0, 127))))

    coords = coords_flat.reshape(B, N, 2)

    adj = pl.pallas_call(
        functools.partial(_edge_kern, N=N),
        grid=(B,),
        in_specs=[
            pl.BlockSpec((1, N, 2), lambda b: (b, 0, 0)),
            pl.BlockSpec((64, 4), lambda b: (0, 0)),
            pl.BlockSpec((1, 64), lambda b: (0, 0)),
            pl.BlockSpec((1, 64), lambda b: (0, 0)),
            pl.BlockSpec((1, 1), lambda b: (0, 0)),
        ],
        out_specs=pl.BlockSpec((1, N, N), lambda b: (b, 0, 0)),
        out_shape=jax.ShapeDtypeStruct((B, N, N), jnp.float32),
    )(coords, ep1_w, ep1_b.reshape(1, 64), ep2_w, ep2_b.reshape(1, 1))

    return coords, adj, cnt[:, 0]


# speedup vs baseline: 14.4495x; 1.1394x over previous
import functools

import jax
import jax.numpy as jnp
from jax.experimental import pallas as pl

_TAPS9 = tuple((dh, dw) for dh in range(3) for dw in range(3))


def _wpool(a, s_ref):
    d0, w, c = a.shape
    za = jax.lax.slice(a, (0, 0, 0), (d0, w - 1, c))
    zb = jax.lax.slice(a, (0, 1, 0), (d0, w, c))
    z = jnp.maximum(za, zb)
    return jax.lax.dot_general(s_ref[:], z, (((1,), (1,)), ((), ())),
                               preferred_element_type=jnp.float32)


def _conv1_kern(x_ref, w_ref, b_ref, s_ref, o_ref, *, ht, W):
    i = pl.program_id(1)
    xt = x_ref[0, pl.ds(i * ht, ht + 2), :]
    sls = [jax.lax.slice(xt, (dh, dw), (dh + ht, dw + W)) for dh, dw in _TAPS9]
    P = jnp.stack(sls, axis=0)
    acc = jax.lax.dot_general(P, w_ref[0], (((0,), (0,)), ((), ())),
                              preferred_element_type=jnp.float32)
    acc = jnp.maximum(acc + b_ref[:], 0.0)
    a = jnp.max(acc.reshape(ht // 2, 2, W, 32), axis=1)
    o_ref[0] = _wpool(a, s_ref)


def _conv_kern(x_ref, w_ref, b_ref, s_ref, o_ref, *, H, W, mode):
    x = x_ref[0]
    cin = x.shape[-1]
    acc = None
    for t, (dh, dw) in enumerate(_TAPS9):
        sl = jax.lax.slice(x, (dh, dw, 0), (dh + H, dw + W, cin))
        p = jax.lax.dot_general(sl.reshape(H * W, cin), w_ref[t],
                                (((1,), (0,)), ((), ())),
                                preferred_element_type=jnp.float32)
        acc = p if acc is None else acc + p
    acc = jnp.maximum(acc + b_ref[:], 0.0)
    if mode == 'mean':
        o_ref[0] = jnp.sum(acc, axis=0, keepdims=True) * (1.0 / (H * W))
    else:
        cout = acc.shape[-1]
        a = jnp.max(acc.reshape(H // 2, 2, W, cout), axis=1)
        o_ref[0] = _wpool(a, s_ref)


def _conv_layer(x, wcol, brow, sel, *, H, W, mode):
    B = x.shape[0]
    cout = wcol.shape[-1]
    if mode == 'mean':
        out_shape = jax.ShapeDtypeStruct((B, 1, cout), jnp.float32)
        out_spec = pl.BlockSpec((1, 1, cout), lambda b: (b, 0, 0))
    else:
        out_shape = jax.ShapeDtypeStruct((B, W // 2, H // 2, cout), jnp.float32)
        out_spec = pl.BlockSpec((1, W // 2, H // 2, cout), lambda b: (b, 0, 0, 0))
    return pl.pallas_call(
        functools.partial(_conv_kern, H=H, W=W, mode=mode),
        grid=(B,),
        in_specs=[
            pl.BlockSpec((1,) + x.shape[1:], lambda b: (b, 0, 0, 0)),
            pl.BlockSpec(wcol.shape, lambda b: (0, 0, 0)),
            pl.BlockSpec(brow.shape, lambda b: (0, 0)),
            pl.BlockSpec(sel.shape, lambda b: (0, 0)),
        ],
        out_specs=out_spec,
        out_shape=out_shape,
    )(x, wcol, brow, sel)


def _heads_kern(gf_ref, n1w_ref, n1b_ref, n2w_ref, n2b_ref,
                c1w_ref, c1b_ref, c2w_ref, c2b_ref, cf_ref, cnt_ref):
    gf = gf_ref[:]
    dn = (((1,), (1,)), ((), ()))
    nh = jnp.maximum(
        jax.lax.dot_general(gf, n1w_ref[:], dn,
                            preferred_element_type=jnp.float32) + n1b_ref[:], 0.0)
    cf_ref[:] = jax.lax.dot_general(nh, n2w_ref[:], dn,
                                    preferred_element_type=jnp.float32) + n2b_ref[:]
    ch = jnp.maximum(
        jax.lax.dot_general(gf, c1w_ref[:], dn,
                            preferred_element_type=jnp.float32) + c1b_ref[:], 0.0)
    cnt_ref[:] = jax.lax.dot_general(ch, c2w_ref[:], dn,
                                     preferred_element_type=jnp.float32) + c2b_ref[:]


def _edge_kern(c_ref, w1_ref, b1_ref, w2_ref, b2_ref, o_ref, *, N):
    c = c_ref[0]
    w1 = w1_ref[:]
    wa = jax.lax.slice(w1, (0, 0), (64, 2))
    wb = jax.lax.slice(w1, (0, 2), (64, 4))
    u = jax.lax.dot_general(c, wa, (((1,), (1,)), ((), ())),
                            preferred_element_type=jnp.float32) + b1_ref[:]
    vT = jax.lax.dot_general(wb, c, (((1,), (1,)), ((), ())),
                             preferred_element_type=jnp.float32)
    E = jnp.zeros((N, N), jnp.float32)
    for k in range(64):
        t = jnp.maximum(u[:, k:k + 1] + vT[k:k + 1, :], 0.0)
        E = E + t * w2_ref[0:1, k:k + 1]
    e = jax.nn.sigmoid(E + b2_ref[0:1, 0:1])
    row = jax.lax.broadcasted_iota(jnp.int32, (N, N), 0)
    col = jax.lax.broadcasted_iota(jnp.int32, (N, N), 1)
    eu = jnp.where(row < col, e, 0.0)
    o_ref[0] = eu + eu.T


def kernel(images, node_masks, c1_w, c1_b, c2_w, c2_b, c3_w, c3_b, c4_w, c4_b,
           np1_w, np1_b, np2_w, np2_b, cp1_w, cp1_b, cp2_w, cp2_b,
           ep1_w, ep1_b, ep2_w, ep2_b):
    B = images.shape[0]
    N = 350

    def col9(w):
        return jnp.transpose(w, (2, 3, 1, 0)).reshape(9, w.shape[1], w.shape[0])

    def col9t(w):
        return jnp.transpose(w, (3, 2, 1, 0)).reshape(9, w.shape[1], w.shape[0])

    def sel(n):
        return (2 * jnp.arange(n // 2)[:, None]
                == jnp.arange(n - 1)[None, :]).astype(jnp.float32)

    w1col = jnp.transpose(c1_w[:, 0], (1, 2, 0)).reshape(1, 9, 32)
    w2col, w3col, w4col = col9t(c2_w), col9(c3_w), col9t(c4_w)
    b1r, b2r, b3r, b4r = (b.reshape(1, -1) for b in (c1_b, c2_b, c3_b, c4_b))
    s1, s2, s3 = sel(224), sel(112), sel(56)

    xpad = jnp.pad(images[:, 0], ((0, 0), (1, 1), (1, 1)))
    ht = 32
    h = pl.pallas_call(
        functools.partial(_conv1_kern, ht=ht, W=224),
        grid=(B, 224 // ht),
        in_specs=[
            pl.BlockSpec((1, 226, 226), lambda b, i: (b, 0, 0)),
            pl.BlockSpec((1, 9, 32), lambda b, i: (0, 0, 0)),
            pl.BlockSpec((1, 32), lambda b, i: (0, 0)),
            pl.BlockSpec((112, 223), lambda b, i: (0, 0)),
        ],
        out_specs=pl.BlockSpec((1, 112, ht // 2, 32), lambda b, i: (b, 0, i, 0)),
        out_shape=jax.ShapeDtypeStruct((B, 112, 112, 32), jnp.float32),
    )(xpad, w1col, b1r, s1)

    h = jnp.pad(h, ((0, 0), (1, 1), (1, 1), (0, 0)))
    h = _conv_layer(h, w2col, b2r, s2, H=112, W=112, mode='pool')
    h = jnp.pad(h, ((0, 0), (1, 1), (1, 1), (0, 0)))
    h = _conv_layer(h, w3col, b3r, s3, H=56, W=56, mode='pool')
    h = jnp.pad(h, ((0, 0), (1, 1), (1, 1), (0, 0)))
    gf = _conv_layer(h, w4col, b4r, s3, H=28, W=28, mode='mean')
    gf = gf.reshape(B, 256)

    coords_flat, cnt = pl.pallas_call(
        _heads_kern,
        grid=(1,),
        in_specs=[pl.BlockSpec(s, lambda g: (0, 0)) for s in
                  ((B, 256), (512, 256), (1, 512), (700, 512), (1, 700),
                   (256, 256), (1, 256), (128, 256), (1, 128))],
        out_specs=[pl.BlockSpec((B, 700), lambda g: (0, 0)),
                   pl.BlockSpec((B, 128), lambda g: (0, 0))],
        out_shape=[jax.ShapeDtypeStruct((B, 700), jnp.float32),
                   jax.ShapeDtypeStruct((B, 128), jnp.float32)],
    )(gf, np1_w, np1_b.reshape(1, 512), np2_w, np2_b.reshape(1, 700),
      cp1_w, cp1_b.reshape(1, 256), jnp.pad(cp2_w, ((0, 127), (0, 0))),
      jnp.pad(cp2_b.reshape(1, 1), ((0, 0), (0, 127))))

    coords = coords_flat.reshape(B, N, 2)

    adj = pl.pallas_call(
        functools.partial(_edge_kern, N=N),
        grid=(B,),
        in_specs=[
            pl.BlockSpec((1, N, 2), lambda b: (b, 0, 0)),
            pl.BlockSpec((64, 4), lambda b: (0, 0)),
            pl.BlockSpec((1, 64), lambda b: (0, 0)),
            pl.BlockSpec((1, 64), lambda b: (0, 0)),
            pl.BlockSpec((1, 1), lambda b: (0, 0)),
        ],
        out_specs=pl.BlockSpec((1, N, N), lambda b: (b, 0, 0)),
        out_shape=jax.ShapeDtypeStruct((B, N, N), jnp.float32),
    )(coords, ep1_w, ep1_b.reshape(1, 64), ep2_w, ep2_b.reshape(1, 1))

    return coords, adj, cnt[:, 0]


# speedup vs baseline: 15.5072x; 1.0732x over previous
import functools

import jax
import jax.numpy as jnp
from jax.experimental import pallas as pl
from jax.experimental.pallas import tpu as pltpu

_PAR1 = pltpu.CompilerParams(dimension_semantics=("parallel",))
_PAR2 = pltpu.CompilerParams(dimension_semantics=("parallel", "parallel"))

_TAPS9 = tuple((dh, dw) for dh in range(3) for dw in range(3))


def _wpool(a, s_ref):
    d0, w, c = a.shape
    za = jax.lax.slice(a, (0, 0, 0), (d0, w - 1, c))
    zb = jax.lax.slice(a, (0, 1, 0), (d0, w, c))
    z = jnp.maximum(za, zb)
    return jax.lax.dot_general(s_ref[:], z, (((1,), (1,)), ((), ())),
                               preferred_element_type=jnp.float32)


def _conv1_kern(x_ref, w_ref, b_ref, s_ref, o_ref, *, ht, W):
    i = pl.program_id(1)
    xt = x_ref[0, pl.ds(i * ht, ht + 2), :]
    sls = [jax.lax.slice(xt, (dh, dw), (dh + ht, dw + W)) for dh, dw in _TAPS9]
    P = jnp.stack(sls, axis=0)
    acc = jax.lax.dot_general(P, w_ref[0], (((0,), (0,)), ((), ())),
                              preferred_element_type=jnp.float32)
    acc = jnp.maximum(acc + b_ref[:], 0.0)
    a = jnp.max(acc.reshape(ht // 2, 2, W, 32), axis=1)
    o_ref[0] = _wpool(a, s_ref)


def _conv_kern(x_ref, w_ref, b_ref, s_ref, o_ref, *, H, W, mode):
    x = x_ref[0]
    cin = x.shape[-1]
    sls = [jax.lax.slice(x, (dh, dw, 0), (dh + H, dw + W, cin)).reshape(
        H * W, cin) for dh, dw in _TAPS9]
    pcat = jnp.concatenate(sls, axis=1)
    acc = jax.lax.dot_general(pcat, w_ref[:], (((1,), (0,)), ((), ())),
                              preferred_element_type=jnp.float32)
    acc = jnp.maximum(acc + b_ref[:], 0.0)
    if mode == 'mean':
        o_ref[0] = jnp.sum(acc, axis=0, keepdims=True) * (1.0 / (H * W))
    else:
        cout = acc.shape[-1]
        a = jnp.max(acc.reshape(H // 2, 2, W, cout), axis=1)
        o_ref[0] = _wpool(a, s_ref)


def _conv_layer(x, wcol, brow, sel, *, H, W, mode):
    B = x.shape[0]
    cout = wcol.shape[-1]
    if mode == 'mean':
        out_shape = jax.ShapeDtypeStruct((B, 1, cout), jnp.float32)
        out_spec = pl.BlockSpec((1, 1, cout), lambda b: (b, 0, 0))
    else:
        out_shape = jax.ShapeDtypeStruct((B, W // 2, H // 2, cout), jnp.float32)
        out_spec = pl.BlockSpec((1, W // 2, H // 2, cout), lambda b: (b, 0, 0, 0))
    return pl.pallas_call(
        functools.partial(_conv_kern, H=H, W=W, mode=mode),
        grid=(B,),
        in_specs=[
            pl.BlockSpec((1,) + x.shape[1:], lambda b: (b, 0, 0, 0)),
            pl.BlockSpec(wcol.shape, lambda b: (0, 0)),
            pl.BlockSpec(brow.shape, lambda b: (0, 0)),
            pl.BlockSpec(sel.shape, lambda b: (0, 0)),
        ],
        out_specs=out_spec,
        out_shape=out_shape,
        compiler_params=_PAR1,
    )(x, wcol, brow, sel)


def _heads_kern(gf_ref, n1w_ref, n1b_ref, n2w_ref, n2b_ref,
                c1w_ref, c1b_ref, c2w_ref, c2b_ref, cf_ref, cnt_ref):
    gf = gf_ref[:]
    dn = (((1,), (1,)), ((), ()))
    nh = jnp.maximum(
        jax.lax.dot_general(gf, n1w_ref[:], dn,
                            preferred_element_type=jnp.float32) + n1b_ref[:], 0.0)
    cf_ref[:] = jax.lax.dot_general(nh, n2w_ref[:], dn,
                                    preferred_element_type=jnp.float32) + n2b_ref[:]
    ch = jnp.maximum(
        jax.lax.dot_general(gf, c1w_ref[:], dn,
                            preferred_element_type=jnp.float32) + c1b_ref[:], 0.0)
    cnt_ref[:] = jax.lax.dot_general(ch, c2w_ref[:], dn,
                                     preferred_element_type=jnp.float32) + c2b_ref[:]


def _edge_kern(c_ref, w1_ref, b1_ref, w2_ref, b2_ref, o_ref, *, N):
    c = c_ref[0]
    w1 = w1_ref[:]
    wa = jax.lax.slice(w1, (0, 0), (64, 2))
    wb = jax.lax.slice(w1, (0, 2), (64, 4))
    u = jax.lax.dot_general(c, wa, (((1,), (1,)), ((), ())),
                            preferred_element_type=jnp.float32) + b1_ref[:]
    vT = jax.lax.dot_general(wb, c, (((1,), (1,)), ((), ())),
                             preferred_element_type=jnp.float32)
    E = jnp.zeros((N, N), jnp.float32)
    for k in range(64):
        t = jnp.maximum(u[:, k:k + 1] + vT[k:k + 1, :], 0.0)
        E = E + t * w2_ref[0:1, k:k + 1]
    e = jax.nn.sigmoid(E + b2_ref[0:1, 0:1])
    row = jax.lax.broadcasted_iota(jnp.int32, (N, N), 0)
    col = jax.lax.broadcasted_iota(jnp.int32, (N, N), 1)
    eu = jnp.where(row < col, e, 0.0)
    o_ref[0] = eu + eu.T


def kernel(images, node_masks, c1_w, c1_b, c2_w, c2_b, c3_w, c3_b, c4_w, c4_b,
           np1_w, np1_b, np2_w, np2_b, cp1_w, cp1_b, cp2_w, cp2_b,
           ep1_w, ep1_b, ep2_w, ep2_b):
    B = images.shape[0]
    N = 350

    def col9(w):
        return jnp.transpose(w, (2, 3, 1, 0)).reshape(9 * w.shape[1], w.shape[0])

    def col9t(w):
        return jnp.transpose(w, (3, 2, 1, 0)).reshape(9 * w.shape[1], w.shape[0])

    def sel(n):
        return (2 * jnp.arange(n // 2)[:, None]
                == jnp.arange(n - 1)[None, :]).astype(jnp.float32)

    w1col = jnp.transpose(c1_w[:, 0], (1, 2, 0)).reshape(1, 9, 32)
    w2col, w3col, w4col = col9t(c2_w), col9(c3_w), col9t(c4_w)
    b1r, b2r, b3r, b4r = (b.reshape(1, -1) for b in (c1_b, c2_b, c3_b, c4_b))
    s1, s2, s3 = sel(224), sel(112), sel(56)

    xpad = jnp.pad(images[:, 0], ((0, 0), (1, 1), (1, 1)))
    ht = 112
    h = pl.pallas_call(
        functools.partial(_conv1_kern, ht=ht, W=224),
        grid=(B, 224 // ht),
        in_specs=[
            pl.BlockSpec((1, 226, 226), lambda b, i: (b, 0, 0)),
            pl.BlockSpec((1, 9, 32), lambda b, i: (0, 0, 0)),
            pl.BlockSpec((1, 32), lambda b, i: (0, 0)),
            pl.BlockSpec((112, 223), lambda b, i: (0, 0)),
        ],
        out_specs=pl.BlockSpec((1, 112, ht // 2, 32), lambda b, i: (b, 0, i, 0)),
        out_shape=jax.ShapeDtypeStruct((B, 112, 112, 32), jnp.float32),
        compiler_params=_PAR2,
    )(xpad, w1col, b1r, s1)

    h = jnp.pad(h, ((0, 0), (1, 1), (1, 1), (0, 0)))
    h = _conv_layer(h, w2col, b2r, s2, H=112, W=112, mode='pool')
    h = jnp.pad(h, ((0, 0), (1, 1), (1, 1), (0, 0)))
    h = _conv_layer(h, w3col, b3r, s3, H=56, W=56, mode='pool')
    h = jnp.pad(h, ((0, 0), (1, 1), (1, 1), (0, 0)))
    gf = _conv_layer(h, w4col, b4r, s3, H=28, W=28, mode='mean')
    gf = gf.reshape(B, 256)

    coords_flat, cnt = pl.pallas_call(
        _heads_kern,
        grid=(1,),
        in_specs=[pl.BlockSpec(s, lambda g: (0, 0)) for s in
                  ((B, 256), (512, 256), (1, 512), (700, 512), (1, 700),
                   (256, 256), (1, 256), (128, 256), (1, 128))],
        out_specs=[pl.BlockSpec((B, 700), lambda g: (0, 0)),
                   pl.BlockSpec((B, 128), lambda g: (0, 0))],
        out_shape=[jax.ShapeDtypeStruct((B, 700), jnp.float32),
                   jax.ShapeDtypeStruct((B, 128), jnp.float32)],
    )(gf, np1_w, np1_b.reshape(1, 512), np2_w, np2_b.reshape(1, 700),
      cp1_w, cp1_b.reshape(1, 256), jnp.pad(cp2_w, ((0, 127), (0, 0))),
      jnp.pad(cp2_b.reshape(1, 1), ((0, 0), (0, 127))))

    coords = coords_flat.reshape(B, N, 2)

    adj = pl.pallas_call(
        functools.partial(_edge_kern, N=N),
        grid=(B,),
        in_specs=[
            pl.BlockSpec((1, N, 2), lambda b: (b, 0, 0)),
            pl.BlockSpec((64, 4), lambda b: (0, 0)),
            pl.BlockSpec((1, 64), lambda b: (0, 0)),
            pl.BlockSpec((1, 64), lambda b: (0, 0)),
            pl.BlockSpec((1, 1), lambda b: (0, 0)),
        ],
        out_specs=pl.BlockSpec((1, N, N), lambda b: (b, 0, 0)),
        out_shape=jax.ShapeDtypeStruct((B, N, N), jnp.float32),
        compiler_params=_PAR1,
    )(coords, ep1_w, ep1_b.reshape(1, 64), ep2_w, ep2_b.reshape(1, 1))

    return coords, adj, cnt[:, 0]
